# baseline (speedup 1.0000x reference)
import functools

import jax
import jax.numpy as jnp
from jax import lax
from jax.experimental import pallas as pl
from jax.experimental.pallas import tpu as pltpu
from jax.experimental.pallas import tpu_sc as plsc

NC = 2
NS = 16
K = 80
NBUF = 2
BN = 1000


def _seg_sum_body(with_count, n, npad, e, d, *refs):
    if with_count:
        (x_hbm, src_hbm, dst3_hbm, zeros_nd, zeros_n, ones_k,
         out_hbm, cnt_hbm,
         src_all, dst_all, ones_v, accum, cnt_acc,
         *bufsems) = refs
        rows = bufsems[:NBUF]
        gsems = bufsems[NBUF:]
    else:
        (x_hbm, src_hbm, dst3_hbm, zeros_nd,
         out_hbm,
         src_all, dst_all, accum,
         *bufsems) = refs
        rows = bufsems[:NBUF]
        gsems = bufsems[NBUF:]

    c = lax.axis_index("c")
    s = lax.axis_index("s")

    edges_per_core = e // NC
    per_sub = edges_per_core // NS
    chunks = per_sub // K
    rows_per_sub = npad // NS
    base = c * edges_per_core + s * per_sub

    pltpu.sync_copy(src_hbm.at[pl.ds(pl.multiple_of(base, 8), per_sub)],
                    src_all)
    pltpu.sync_copy(dst3_hbm.at[pl.ds(base // K, chunks)], dst_all)

    r0 = s * rows_per_sub
    pltpu.sync_copy(zeros_nd.at[pl.ds(r0, rows_per_sub)],
                    accum.at[pl.ds(r0, rows_per_sub)])
    if with_count:
        @pl.when(s == 0)
        def _():
            pltpu.sync_copy(zeros_n, cnt_acc)
        pltpu.sync_copy(ones_k, ones_v)
    plsc.subcore_barrier()

    def gather(i, b):
        idx = src_all.at[pl.ds(pl.multiple_of(i * K, 8), K)]
        pltpu.async_copy(x_hbm.at[idx], rows[b], gsems[b])

    def wait_gather(i, b):
        idx = src_all.at[pl.ds(pl.multiple_of(i * K, 8), K)]
        pltpu.make_async_copy(x_hbm.at[idx], rows[b], gsems[b]).wait()

    def scatter(i, b):
        didx = dst_all.at[i, 0]
        pltpu.sync_copy(rows[b], accum.at[didx], add=True)
        if with_count:
            pltpu.sync_copy(ones_v, cnt_acc.at[didx], add=True)

    gather(0, 0)

    def outer(j, carry):
        gather(2 * j + 1, 1)
        wait_gather(2 * j, 0)
        scatter(2 * j, 0)
        gather(2 * j + 2, 0)
        wait_gather(2 * j + 1, 1)
        scatter(2 * j + 1, 1)
        return carry

    lax.fori_loop(0, (chunks - 1) // 2, outer, 0)
    if chunks % 2 == 1:
        wait_gather(chunks - 1, 0)
        scatter(chunks - 1, 0)
    else:
        gather(chunks - 1, 1)
        wait_gather(chunks - 2, 0)
        scatter(chunks - 2, 0)
        wait_gather(chunks - 1, 1)
        scatter(chunks - 1, 1)

    plsc.subcore_barrier()
    pltpu.sync_copy(accum.at[pl.ds(r0, rows_per_sub)],
                    out_hbm.at[c, pl.ds(r0, rows_per_sub)])
    if with_count:
        @pl.when(s == 0)
        def _():
            pltpu.sync_copy(cnt_acc, cnt_hbm.at[c, 0])


@functools.lru_cache(maxsize=None)
def _make_seg_sum(n, npad, e, d, with_count):
    assert e % (NC * NS * K) == 0 and npad % (NS * 8) == 0
    per_sub = e // (NC * NS)
    chunks = per_sub // K
    mesh = plsc.VectorSubcoreMesh(core_axis_name="c", subcore_axis_name="s",
                                  num_cores=NC, num_subcores=NS)
    out_type = [jax.ShapeDtypeStruct((NC, npad, d), jnp.float32)]
    scratch = [
        pltpu.VMEM((per_sub,), jnp.int32),
        pltpu.VMEM((chunks, 1, K), jnp.int32),
    ]
    if with_count:
        out_type.append(jax.ShapeDtypeStruct((NC, 1, npad), jnp.float32))
        scratch.append(pltpu.VMEM((K,), jnp.float32))
    scratch.append(pltpu.VMEM_SHARED((npad, d), jnp.float32))
    if with_count:
        scratch.append(pltpu.VMEM_SHARED((npad,), jnp.float32))
    scratch += [pltpu.VMEM((K, d), jnp.float32)] * NBUF
    scratch += [pltpu.SemaphoreType.DMA] * NBUF
    return pl.kernel(
        functools.partial(_seg_sum_body, with_count, n, npad, e, d),
        out_type=out_type if with_count else out_type[0],
        mesh=mesh,
        scratch_types=scratch,
    )


def _tc_layer_body(act, sums_ref, cnt_ref, x_ref, wl_ref, wr_ref, out_ref):
    ssum = sums_ref[0] + sums_ref[1]
    cnt = cnt_ref[0, 0, 0, :] + cnt_ref[1, 0, 0, :]
    inv = 1.0 / jnp.maximum(cnt, 1.0)
    mean = ssum * inv[:, None]
    z = lax.dot_general(mean, wl_ref[...], (((1,), (1,)), ((), ())),
                        preferred_element_type=jnp.float32)
    z = z + lax.dot_general(x_ref[...], wr_ref[...], (((1,), (1,)), ((), ())),
                            preferred_element_type=jnp.float32)
    if act == "relu":
        out_ref[...] = jnp.maximum(z, 0.0)
    else:
        m = jnp.max(z, axis=1, keepdims=True)
        out_ref[...] = (z - m) - jnp.log(
            jnp.sum(jnp.exp(z - m), axis=1, keepdims=True))


@functools.lru_cache(maxsize=None)
def _make_tc_layer(n, npad, d, act):
    assert n % BN == 0
    nb = n // BN
    return pl.pallas_call(
        functools.partial(_tc_layer_body, act),
        grid=(nb,),
        in_specs=[
            pl.BlockSpec((NC, BN, d), lambda i: (0, i, 0)),
            pl.BlockSpec((NC, 1, 1, BN), lambda i: (0, i, 0, 0)),
            pl.BlockSpec((BN, d), lambda i: (i, 0)),
            pl.BlockSpec((d, d), lambda i: (0, 0)),
            pl.BlockSpec((d, d), lambda i: (0, 0)),
        ],
        out_specs=pl.BlockSpec((BN, d), lambda i: (i, 0)),
        out_shape=jax.ShapeDtypeStruct((n, d), jnp.float32),
    )


def kernel(x, edge_index, Wl1, Wr1, Wl2, Wr2):
    x = x.astype(jnp.float32)
    n, d = x.shape
    e = edge_index.shape[1]
    npad = ((n + NS * 8 - 1) // (NS * 8)) * NS * 8
    grain = NC * NS * K * 2
    ep = ((e + grain - 1) // grain) * grain
    src = jnp.concatenate(
        [edge_index[0], jnp.zeros((ep - e,), jnp.int32)])
    dummy_dst = n + jnp.arange(ep - e, dtype=jnp.int32) % max(npad - n, 1)
    dst3 = jnp.concatenate(
        [edge_index[1], dummy_dst]).reshape(ep // K, 1, K)
    zeros_nd = jnp.zeros((npad, d), jnp.float32)
    zeros_n = jnp.zeros((npad,), jnp.float32)
    ones_k = jnp.ones((K,), jnp.float32)

    sums1, cnt = _make_seg_sum(n, npad, ep, d, True)(
        x, src, dst3, zeros_nd, zeros_n, ones_k)
    cnt4 = cnt[:, 0, :n].reshape(NC, n // BN, 1, BN)
    h = _make_tc_layer(n, npad, d, "relu")(sums1, cnt4, x, Wl1, Wr1)
    sums2 = _make_seg_sum(n, npad, ep, d, False)(h, src, dst3, zeros_nd)
    out = _make_tc_layer(n, npad, d, "logsoftmax")(sums2, cnt4, h, Wl2, Wr2)
    return out


# speedup vs baseline: 1.7138x; 1.7138x over previous
import functools

import jax
import jax.numpy as jnp
from jax import lax
from jax.experimental import pallas as pl
from jax.experimental.pallas import tpu as pltpu
from jax.experimental.pallas import tpu_sc as plsc

NC = 2
NS = 16
K = 80
NBUF = 2
BN = 1000


def _seg_sum_body(with_count, n, npad, e, d, *refs):
    if with_count:
        (x_hbm, src_hbm, dst3_hbm, zeros_nd, zeros_n, ones_k,
         out_hbm, cnt_hbm,
         src_all, dst_all, ones_v, accum, cnt_acc,
         *bufsems) = refs
        rows = bufsems[:NBUF]
        gsems = bufsems[NBUF:]
    else:
        (x_hbm, src_hbm, dst3_hbm, zeros_nd,
         out_hbm,
         src_all, dst_all, accum,
         *bufsems) = refs
        rows = bufsems[:NBUF]
        gsems = bufsems[NBUF:]

    c = lax.axis_index("c")
    s = lax.axis_index("s")

    edges_per_core = e // NC
    per_sub = edges_per_core // NS
    chunks = per_sub // K
    rows_per_sub = npad // NS
    base = c * edges_per_core + s * per_sub

    pltpu.sync_copy(src_hbm.at[pl.ds(pl.multiple_of(base, 8), per_sub)],
                    src_all)
    pltpu.sync_copy(dst3_hbm.at[pl.ds(base // K, chunks)], dst_all)

    r0 = s * rows_per_sub
    pltpu.sync_copy(zeros_nd.at[pl.ds(r0, rows_per_sub)],
                    accum.at[pl.ds(r0, rows_per_sub)])
    if with_count:
        @pl.when(s == 0)
        def _():
            pltpu.sync_copy(zeros_n, cnt_acc)
        pltpu.sync_copy(ones_k, ones_v)
    plsc.subcore_barrier()

    def gather(i, b):
        idx = src_all.at[pl.ds(pl.multiple_of(i * K, 8), K)]
        pltpu.async_copy(x_hbm.at[idx], rows[b], gsems[b])

    def wait_gather(i, b):
        idx = src_all.at[pl.ds(pl.multiple_of(i * K, 8), K)]
        pltpu.make_async_copy(x_hbm.at[idx], rows[b], gsems[b]).wait()

    def scatter(i, b):
        didx = dst_all.at[i, 0]
        pltpu.sync_copy(rows[b], accum.at[didx], add=True)
        if with_count:
            pltpu.sync_copy(ones_v, cnt_acc.at[didx], add=True)

    gather(0, 0)

    def outer(j, carry):
        gather(2 * j + 1, 1)
        wait_gather(2 * j, 0)
        scatter(2 * j, 0)
        gather(2 * j + 2, 0)
        wait_gather(2 * j + 1, 1)
        scatter(2 * j + 1, 1)
        return carry

    lax.fori_loop(0, (chunks - 1) // 2, outer, 0)
    if chunks % 2 == 1:
        wait_gather(chunks - 1, 0)
        scatter(chunks - 1, 0)
    else:
        gather(chunks - 1, 1)
        wait_gather(chunks - 2, 0)
        scatter(chunks - 2, 0)
        wait_gather(chunks - 1, 1)
        scatter(chunks - 1, 1)

    plsc.subcore_barrier()
    pltpu.sync_copy(accum.at[pl.ds(r0, rows_per_sub)],
                    out_hbm.at[c, pl.ds(r0, rows_per_sub)])
    if with_count:
        @pl.when(s == 0)
        def _():
            pltpu.sync_copy(cnt_acc, cnt_hbm.at[c, 0])


@functools.lru_cache(maxsize=None)
def _make_seg_sum(n, npad, e, d, with_count):
    assert e % (NC * NS * K) == 0 and npad % (NS * 8) == 0
    per_sub = e // (NC * NS)
    chunks = per_sub // K
    mesh = plsc.VectorSubcoreMesh(core_axis_name="c", subcore_axis_name="s",
                                  num_cores=NC, num_subcores=NS)
    out_type = [jax.ShapeDtypeStruct((NC, npad, d), jnp.float32)]
    scratch = [
        pltpu.VMEM((per_sub,), jnp.int32),
        pltpu.VMEM((chunks, 1, K), jnp.int32),
    ]
    if with_count:
        out_type.append(jax.ShapeDtypeStruct((NC, 1, npad), jnp.float32))
        scratch.append(pltpu.VMEM((K,), jnp.float32))
    scratch.append(pltpu.VMEM_SHARED((npad, d), jnp.float32))
    if with_count:
        scratch.append(pltpu.VMEM_SHARED((npad,), jnp.float32))
    scratch += [pltpu.VMEM((K, d), jnp.float32)] * NBUF
    scratch += [pltpu.SemaphoreType.DMA] * NBUF
    return pl.kernel(
        functools.partial(_seg_sum_body, with_count, n, npad, e, d),
        out_type=out_type if with_count else out_type[0],
        mesh=mesh,
        scratch_types=scratch,
    )


def _tc_layer_body(act, sums_ref, cnt_ref, x_ref, wl_ref, wr_ref, out_ref):
    ssum = sums_ref[0] + sums_ref[1]
    cnt = cnt_ref[0, 0, 0, :] + cnt_ref[1, 0, 0, :]
    inv = 1.0 / jnp.maximum(cnt, 1.0)
    mean = ssum * inv[:, None]
    z = lax.dot_general(mean, wl_ref[...], (((1,), (1,)), ((), ())),
                        preferred_element_type=jnp.float32)
    z = z + lax.dot_general(x_ref[...], wr_ref[...], (((1,), (1,)), ((), ())),
                            preferred_element_type=jnp.float32)
    if act == "relu":
        out_ref[...] = jnp.maximum(z, 0.0)
    else:
        m = jnp.max(z, axis=1, keepdims=True)
        out_ref[...] = (z - m) - jnp.log(
            jnp.sum(jnp.exp(z - m), axis=1, keepdims=True))


@functools.lru_cache(maxsize=None)
def _make_tc_layer(n, npad, d, act):
    assert n % BN == 0
    nb = n // BN
    return pl.pallas_call(
        functools.partial(_tc_layer_body, act),
        grid=(nb,),
        in_specs=[
            pl.BlockSpec((NC, BN, d), lambda i: (0, i, 0)),
            pl.BlockSpec((NC, 1, 1, BN), lambda i: (0, i, 0, 0)),
            pl.BlockSpec((BN, d), lambda i: (i, 0)),
            pl.BlockSpec((d, d), lambda i: (0, 0)),
            pl.BlockSpec((d, d), lambda i: (0, 0)),
        ],
        out_specs=pl.BlockSpec((BN, d), lambda i: (i, 0)),
        out_shape=jax.ShapeDtypeStruct((n, d), jnp.float32),
    )


def kernel(x, edge_index, Wl1, Wr1, Wl2, Wr2):
    x = x.astype(jnp.float32)
    n, d = x.shape
    e = edge_index.shape[1]
    npad = ((n + NS * 8 - 1) // (NS * 8)) * NS * 8
    grain = NC * NS * K * 2
    ep = ((e + grain - 1) // grain) * grain
    src = jnp.concatenate(
        [edge_index[0], jnp.arange(ep - e, dtype=jnp.int32) % n])
    dummy_dst = n + jnp.arange(ep - e, dtype=jnp.int32) % max(npad - n, 1)
    dst3 = jnp.concatenate(
        [edge_index[1], dummy_dst]).reshape(ep // K, 1, K)
    zeros_nd = jnp.zeros((npad, d), jnp.float32)
    zeros_n = jnp.zeros((npad,), jnp.float32)
    ones_k = jnp.ones((K,), jnp.float32)

    sums1, cnt = _make_seg_sum(n, npad, ep, d, True)(
        x, src, dst3, zeros_nd, zeros_n, ones_k)
    cnt4 = cnt[:, 0, :n].reshape(NC, n // BN, 1, BN)
    h = _make_tc_layer(n, npad, d, "relu")(sums1, cnt4, x, Wl1, Wr1)
    sums2 = _make_seg_sum(n, npad, ep, d, False)(h, src, dst3, zeros_nd)
    out = _make_tc_layer(n, npad, d, "logsoftmax")(sums2, cnt4, h, Wl2, Wr2)
    return out


# speedup vs baseline: 1.7938x; 1.0467x over previous
import functools

import jax
import jax.numpy as jnp
from jax import lax
from jax.experimental import pallas as pl
from jax.experimental.pallas import tpu as pltpu
from jax.experimental.pallas import tpu_sc as plsc

NC = 2
NS = 16
K = 96
NBUF = 2
BN = 1000


def _seg_sum_body(with_count, n, npad, e, d, *refs):
    if with_count:
        (x_hbm, src_hbm, dst3_hbm, zeros_nd, zeros_n, ones_k,
         out_hbm, cnt_hbm,
         src_all, dst_all, ones_v, accum, cnt_acc,
         *bufsems) = refs
        rows = bufsems[:NBUF]
        gsems = bufsems[NBUF:]
    else:
        (x_hbm, src_hbm, dst3_hbm, zeros_nd,
         out_hbm,
         src_all, dst_all, accum,
         *bufsems) = refs
        rows = bufsems[:NBUF]
        gsems = bufsems[NBUF:]

    c = lax.axis_index("c")
    s = lax.axis_index("s")

    edges_per_core = e // NC
    per_sub = edges_per_core // NS
    chunks = per_sub // K
    rows_per_sub = npad // NS
    base = c * edges_per_core + s * per_sub

    pltpu.sync_copy(src_hbm.at[pl.ds(pl.multiple_of(base, 8), per_sub)],
                    src_all)
    pltpu.sync_copy(dst3_hbm.at[pl.ds(base // K, chunks)], dst_all)

    r0 = s * rows_per_sub
    pltpu.sync_copy(zeros_nd.at[pl.ds(r0, rows_per_sub)],
                    accum.at[pl.ds(r0, rows_per_sub)])
    if with_count:
        @pl.when(s == 0)
        def _():
            pltpu.sync_copy(zeros_n, cnt_acc)
        pltpu.sync_copy(ones_k, ones_v)
    plsc.subcore_barrier()

    def gather(i, b):
        idx = src_all.at[pl.ds(pl.multiple_of(i * K, 8), K)]
        pltpu.async_copy(x_hbm.at[idx], rows[b], gsems[b])

    def wait_gather(i, b):
        idx = src_all.at[pl.ds(pl.multiple_of(i * K, 8), K)]
        pltpu.make_async_copy(x_hbm.at[idx], rows[b], gsems[b]).wait()

    def scatter(i, b):
        didx = dst_all.at[i, 0]
        pltpu.sync_copy(rows[b], accum.at[didx], add=True)
        if with_count:
            pltpu.sync_copy(ones_v, cnt_acc.at[didx], add=True)

    gather(0, 0)

    def outer(j, carry):
        gather(2 * j + 1, 1)
        wait_gather(2 * j, 0)
        scatter(2 * j, 0)
        gather(2 * j + 2, 0)
        wait_gather(2 * j + 1, 1)
        scatter(2 * j + 1, 1)
        return carry

    lax.fori_loop(0, (chunks - 1) // 2, outer, 0)
    if chunks % 2 == 1:
        wait_gather(chunks - 1, 0)
        scatter(chunks - 1, 0)
    else:
        gather(chunks - 1, 1)
        wait_gather(chunks - 2, 0)
        scatter(chunks - 2, 0)
        wait_gather(chunks - 1, 1)
        scatter(chunks - 1, 1)

    plsc.subcore_barrier()
    pltpu.sync_copy(accum.at[pl.ds(r0, rows_per_sub)],
                    out_hbm.at[c, pl.ds(r0, rows_per_sub)])
    if with_count:
        @pl.when(s == 0)
        def _():
            pltpu.sync_copy(cnt_acc, cnt_hbm.at[c, 0])


@functools.lru_cache(maxsize=None)
def _make_seg_sum(n, npad, e, d, with_count):
    assert e % (NC * NS * K) == 0 and npad % (NS * 8) == 0
    per_sub = e // (NC * NS)
    chunks = per_sub // K
    mesh = plsc.VectorSubcoreMesh(core_axis_name="c", subcore_axis_name="s",
                                  num_cores=NC, num_subcores=NS)
    out_type = [jax.ShapeDtypeStruct((NC, npad, d), jnp.float32)]
    scratch = [
        pltpu.VMEM((per_sub,), jnp.int32),
        pltpu.VMEM((chunks, 1, K), jnp.int32),
    ]
    if with_count:
        out_type.append(jax.ShapeDtypeStruct((NC, 1, npad), jnp.float32))
        scratch.append(pltpu.VMEM((K,), jnp.float32))
    scratch.append(pltpu.VMEM_SHARED((npad, d), jnp.float32))
    if with_count:
        scratch.append(pltpu.VMEM_SHARED((npad,), jnp.float32))
    scratch += [pltpu.VMEM((K, d), jnp.float32)] * NBUF
    scratch += [pltpu.SemaphoreType.DMA] * NBUF
    return pl.kernel(
        functools.partial(_seg_sum_body, with_count, n, npad, e, d),
        out_type=out_type if with_count else out_type[0],
        mesh=mesh,
        scratch_types=scratch,
    )


def _tc_layer_body(act, sums_ref, cnt_ref, x_ref, wl_ref, wr_ref, out_ref):
    ssum = sums_ref[0] + sums_ref[1]
    cnt = cnt_ref[0, 0, 0, :] + cnt_ref[1, 0, 0, :]
    inv = 1.0 / jnp.maximum(cnt, 1.0)
    mean = ssum * inv[:, None]
    z = lax.dot_general(mean, wl_ref[...], (((1,), (1,)), ((), ())),
                        preferred_element_type=jnp.float32)
    z = z + lax.dot_general(x_ref[...], wr_ref[...], (((1,), (1,)), ((), ())),
                            preferred_element_type=jnp.float32)
    if act == "relu":
        out_ref[...] = jnp.maximum(z, 0.0)
    else:
        m = jnp.max(z, axis=1, keepdims=True)
        out_ref[...] = (z - m) - jnp.log(
            jnp.sum(jnp.exp(z - m), axis=1, keepdims=True))


@functools.lru_cache(maxsize=None)
def _make_tc_layer(n, npad, d, act):
    assert n % BN == 0
    nb = n // BN
    return pl.pallas_call(
        functools.partial(_tc_layer_body, act),
        grid=(nb,),
        in_specs=[
            pl.BlockSpec((NC, BN, d), lambda i: (0, i, 0)),
            pl.BlockSpec((NC, 1, 1, BN), lambda i: (0, i, 0, 0)),
            pl.BlockSpec((BN, d), lambda i: (i, 0)),
            pl.BlockSpec((d, d), lambda i: (0, 0)),
            pl.BlockSpec((d, d), lambda i: (0, 0)),
        ],
        out_specs=pl.BlockSpec((BN, d), lambda i: (i, 0)),
        out_shape=jax.ShapeDtypeStruct((n, d), jnp.float32),
    )


def kernel(x, edge_index, Wl1, Wr1, Wl2, Wr2):
    x = x.astype(jnp.float32)
    n, d = x.shape
    e = edge_index.shape[1]
    npad = ((n + NS * 8 - 1) // (NS * 8)) * NS * 8
    grain = NC * NS * K
    ep = ((e + grain - 1) // grain) * grain
    src = jnp.concatenate(
        [edge_index[0], jnp.arange(ep - e, dtype=jnp.int32) % n])
    dummy_dst = n + jnp.arange(ep - e, dtype=jnp.int32) % max(npad - n, 1)
    dst3 = jnp.concatenate(
        [edge_index[1], dummy_dst]).reshape(ep // K, 1, K)
    zeros_nd = jnp.zeros((npad, d), jnp.float32)
    zeros_n = jnp.zeros((npad,), jnp.float32)
    ones_k = jnp.ones((K,), jnp.float32)

    sums1, cnt = _make_seg_sum(n, npad, ep, d, True)(
        x, src, dst3, zeros_nd, zeros_n, ones_k)
    cnt4 = cnt[:, 0, :n].reshape(NC, n // BN, 1, BN)
    h = _make_tc_layer(n, npad, d, "relu")(sums1, cnt4, x, Wl1, Wr1)
    sums2 = _make_seg_sum(n, npad, ep, d, False)(h, src, dst3, zeros_nd)
    out = _make_tc_layer(n, npad, d, "logsoftmax")(sums2, cnt4, h, Wl2, Wr2)
    return out


# speedup vs baseline: 1.8586x; 1.0361x over previous
import functools

import jax
import jax.numpy as jnp
from jax import lax
from jax.experimental import pallas as pl
from jax.experimental.pallas import tpu as pltpu
from jax.experimental.pallas import tpu_sc as plsc

NC = 2
NS = 16
K = 96
NBUF = 2
BN = 2000


def _seg_sum_body(with_count, n, npad, e, d, *refs):
    if with_count:
        (x_hbm, src_hbm, dst3_hbm, zeros_nd, zeros_n, ones_k,
         out_hbm, cnt_hbm,
         src_all, dst_all, ones_v, accum, cnt_acc,
         *bufsems) = refs
        rows = bufsems[:NBUF]
        gsems = bufsems[NBUF:]
    else:
        (x_hbm, src_hbm, dst3_hbm, zeros_nd,
         out_hbm,
         src_all, dst_all, accum,
         *bufsems) = refs
        rows = bufsems[:NBUF]
        gsems = bufsems[NBUF:]

    c = lax.axis_index("c")
    s = lax.axis_index("s")

    edges_per_core = e // NC
    per_sub = edges_per_core // NS
    chunks = per_sub // K
    rows_per_sub = npad // NS
    base = c * edges_per_core + s * per_sub

    r0 = s * rows_per_sub
    zero_desc = pltpu.make_async_copy(zeros_nd.at[pl.ds(r0, rows_per_sub)],
                                      accum.at[pl.ds(r0, rows_per_sub)],
                                      gsems[0])
    zero_desc.start()
    pltpu.sync_copy(src_hbm.at[pl.ds(pl.multiple_of(base, 8), per_sub)],
                    src_all)
    pltpu.sync_copy(dst3_hbm.at[pl.ds(base // K, chunks)], dst_all)
    if with_count:
        @pl.when(s == 0)
        def _():
            pltpu.sync_copy(zeros_n, cnt_acc)
        pltpu.sync_copy(ones_k, ones_v)
    zero_desc.wait()
    plsc.subcore_barrier()

    def gather(i, b):
        idx = src_all.at[pl.ds(pl.multiple_of(i * K, 8), K)]
        pltpu.async_copy(x_hbm.at[idx], rows[b], gsems[b])

    def wait_gather(i, b):
        idx = src_all.at[pl.ds(pl.multiple_of(i * K, 8), K)]
        pltpu.make_async_copy(x_hbm.at[idx], rows[b], gsems[b]).wait()

    def scatter(i, b):
        didx = dst_all.at[i, 0]
        pltpu.sync_copy(rows[b], accum.at[didx], add=True)
        if with_count:
            pltpu.sync_copy(ones_v, cnt_acc.at[didx], add=True)

    gather(0, 0)

    def outer(j, carry):
        gather(2 * j + 1, 1)
        wait_gather(2 * j, 0)
        scatter(2 * j, 0)
        gather(2 * j + 2, 0)
        wait_gather(2 * j + 1, 1)
        scatter(2 * j + 1, 1)
        return carry

    lax.fori_loop(0, (chunks - 1) // 2, outer, 0)
    if chunks % 2 == 1:
        wait_gather(chunks - 1, 0)
        scatter(chunks - 1, 0)
    else:
        gather(chunks - 1, 1)
        wait_gather(chunks - 2, 0)
        scatter(chunks - 2, 0)
        wait_gather(chunks - 1, 1)
        scatter(chunks - 1, 1)

    plsc.subcore_barrier()
    pltpu.sync_copy(accum.at[pl.ds(r0, rows_per_sub)],
                    out_hbm.at[c, pl.ds(r0, rows_per_sub)])
    if with_count:
        @pl.when(s == 0)
        def _():
            pltpu.sync_copy(cnt_acc, cnt_hbm.at[c, 0])


@functools.lru_cache(maxsize=None)
def _make_seg_sum(n, npad, e, d, with_count):
    assert e % (NC * NS * K) == 0 and npad % (NS * 8) == 0
    per_sub = e // (NC * NS)
    chunks = per_sub // K
    mesh = plsc.VectorSubcoreMesh(core_axis_name="c", subcore_axis_name="s",
                                  num_cores=NC, num_subcores=NS)
    out_type = [jax.ShapeDtypeStruct((NC, npad, d), jnp.float32)]
    scratch = [
        pltpu.VMEM((per_sub,), jnp.int32),
        pltpu.VMEM((chunks, 1, K), jnp.int32),
    ]
    if with_count:
        out_type.append(jax.ShapeDtypeStruct((NC, 1, npad), jnp.float32))
        scratch.append(pltpu.VMEM((K,), jnp.float32))
    scratch.append(pltpu.VMEM_SHARED((npad, d), jnp.float32))
    if with_count:
        scratch.append(pltpu.VMEM_SHARED((npad,), jnp.float32))
    scratch += [pltpu.VMEM((K, d), jnp.float32)] * NBUF
    scratch += [pltpu.SemaphoreType.DMA] * NBUF
    return pl.kernel(
        functools.partial(_seg_sum_body, with_count, n, npad, e, d),
        out_type=out_type if with_count else out_type[0],
        mesh=mesh,
        scratch_types=scratch,
    )


def _tc_layer_body(act, sums_ref, cnt_ref, x_ref, wl_ref, wr_ref, out_ref):
    ssum = sums_ref[0] + sums_ref[1]
    cnt = cnt_ref[0, 0, 0, :] + cnt_ref[1, 0, 0, :]
    inv = 1.0 / jnp.maximum(cnt, 1.0)
    mean = ssum * inv[:, None]
    z = lax.dot_general(mean, wl_ref[...], (((1,), (1,)), ((), ())),
                        preferred_element_type=jnp.float32)
    z = z + lax.dot_general(x_ref[...], wr_ref[...], (((1,), (1,)), ((), ())),
                            preferred_element_type=jnp.float32)
    if act == "relu":
        out_ref[...] = jnp.maximum(z, 0.0)
    else:
        m = jnp.max(z, axis=1, keepdims=True)
        out_ref[...] = (z - m) - jnp.log(
            jnp.sum(jnp.exp(z - m), axis=1, keepdims=True))


@functools.lru_cache(maxsize=None)
def _make_tc_layer(n, npad, d, act):
    assert n % BN == 0
    nb = n // BN
    return pl.pallas_call(
        functools.partial(_tc_layer_body, act),
        grid=(nb,),
        in_specs=[
            pl.BlockSpec((NC, BN, d), lambda i: (0, i, 0)),
            pl.BlockSpec((NC, 1, 1, BN), lambda i: (0, i, 0, 0)),
            pl.BlockSpec((BN, d), lambda i: (i, 0)),
            pl.BlockSpec((d, d), lambda i: (0, 0)),
            pl.BlockSpec((d, d), lambda i: (0, 0)),
        ],
        out_specs=pl.BlockSpec((BN, d), lambda i: (i, 0)),
        out_shape=jax.ShapeDtypeStruct((n, d), jnp.float32),
    )


def kernel(x, edge_index, Wl1, Wr1, Wl2, Wr2):
    x = x.astype(jnp.float32)
    n, d = x.shape
    e = edge_index.shape[1]
    npad = ((n + NS * 8 - 1) // (NS * 8)) * NS * 8
    grain = NC * NS * K
    ep = ((e + grain - 1) // grain) * grain
    src = jnp.concatenate(
        [edge_index[0], jnp.arange(ep - e, dtype=jnp.int32) % n])
    dummy_dst = n + jnp.arange(ep - e, dtype=jnp.int32) % max(npad - n, 1)
    dst3 = jnp.concatenate(
        [edge_index[1], dummy_dst]).reshape(ep // K, 1, K)
    zeros_nd = jnp.zeros((npad, d), jnp.float32)
    zeros_n = jnp.zeros((npad,), jnp.float32)
    ones_k = jnp.ones((K,), jnp.float32)

    sums1, cnt = _make_seg_sum(n, npad, ep, d, True)(
        x, src, dst3, zeros_nd, zeros_n, ones_k)
    cnt4 = cnt[:, 0, :n].reshape(NC, n // BN, 1, BN)
    h = _make_tc_layer(n, npad, d, "relu")(sums1, cnt4, x, Wl1, Wr1)
    sums2 = _make_seg_sum(n, npad, ep, d, False)(h, src, dst3, zeros_nd)
    out = _make_tc_layer(n, npad, d, "logsoftmax")(sums2, cnt4, h, Wl2, Wr2)
    return out
